# Initial kernel scaffold; baseline (speedup 1.0000x reference)
import jax
import jax.numpy as jnp
from jax.experimental import pallas as pl


def kernel(x, range_weight):
    raise NotImplementedError("write your pallas kernel here")


# speedup vs baseline: 6.2015x; 6.2015x over previous
import jax
import jax.numpy as jnp
from jax.experimental import pallas as pl

_SEARCH = 7
_PAD = (_SEARCH - 1) // 2
_KNN = 9
_NE = 2
_SD = _SEARCH * _SEARCH


def _make_body(R, H, W, stem, n_chan):
    INT_MAX = 0x7FFFFFFF

    def body(xp_ref, w_ref, out_ref):
        r = pl.program_id(1)
        row0 = r * R
        chans = [xp_ref[0, c, pl.ds(row0, R + 2 * _PAD), :] for c in range(n_chan)]

        def center(a):
            return a[_PAD:_PAD + R, _PAD:_PAD + W]

        fur = jnp.stack(
            [chans[0][di:di + R, dj:dj + W]
             for di in range(_SEARCH) for dj in range(_SEARCH)], axis=0)
        fup = [jnp.stack(
            [chans[2 + c][di:di + R, dj:dj + W]
             for di in range(_SEARCH) for dj in range(_SEARCH)], axis=0)
            for c in range(3)]

        iota = jax.lax.broadcasted_iota(jnp.int32, (_SD, R, W), 0)
        slots = []
        for e in range(_NE):
            dsq = None
            for c in range(3):
                npc = center(chans[2 + 3 * e + c])
                d = fup[c] - npc[None, :, :]
                d = d * d
                dsq = d if dsq is None else dsq + d
            keys = jax.lax.bitcast_convert_type(dsq, jnp.int32)
            for _ in range(_KNN):
                m = jnp.min(keys, axis=0, keepdims=True)
                tie = keys == m
                idxs = jnp.min(jnp.where(tie, iota, _SD), axis=0, keepdims=True)
                one = jnp.logical_and(tie, iota == idxs)
                slots.append(jnp.sum(jnp.where(one, fur, 0.0), axis=0))
                keys = jnp.where(one, jnp.int32(INT_MAX), keys)
            slots.append(center(chans[e]))

        u = jnp.stack(slots, axis=0).reshape(_KNN * _NE + _NE, R * W)
        o = jax.lax.dot_general(
            w_ref[...], u, (((1,), (0,)), ((), ())),
            preferred_element_type=jnp.float32)
        o = o.reshape(stem, R, W)
        out_ref[0] = jnp.where(o >= 0, o, 0.01 * o)

    return body


def kernel(x, range_weight):
    B, C, H, W = x.shape
    stem = range_weight.shape[1]
    k_total = range_weight.shape[2]
    R = 8
    xp = jnp.pad(x, ((0, 0), (0, 0), (_PAD, _PAD), (_PAD, _PAD)))
    body = _make_body(R, H, W, stem, C)
    out = pl.pallas_call(
        body,
        grid=(B, H // R),
        in_specs=[
            pl.BlockSpec((1, C, H + 2 * _PAD, W + 2 * _PAD),
                         lambda b, r: (b, 0, 0, 0)),
            pl.BlockSpec((stem, k_total), lambda b, r: (0, 0)),
        ],
        out_specs=pl.BlockSpec((1, stem, R, W), lambda b, r: (b, 0, r, 0)),
        out_shape=jax.ShapeDtypeStruct((B, stem, H, W), jnp.float32),
    )(xp, range_weight[0])
    return out
